# baseline (speedup 1.0000x reference)
import functools

import jax
import jax.numpy as jnp
from jax import lax
from jax.experimental import pallas as pl
from jax.experimental.pallas import tpu as pltpu
from jax.experimental.pallas import tpu_sc as plsc

N = 10000
E = 320000
D = 128
DEG_W = 16
D_OUT = 40
D_OUT_PAD = 48

NC = 2
NS = 16
NT = NC * NS
CHUNK = 128
CPT = 80
EPT = E // NT
EPT_PAD = CPT * CHUNK

NROWS_ACC = N + 16
ROWS_A = 624
TAIL0 = NS * ROWS_A
WTAIL = N - TAIL0
ZTAIL = NROWS_ACC - TAIL0
Z_LENS = (128, 128, 128, 128, 112)

NBUF = 4
NIDX = 8
STEADY0 = 4
STEADY1 = CPT - 4

A_NBUF = 3
A_NIDX = 6
A_STEADY0 = 3


def _sc_mesh():
    return plsc.VectorSubcoreMesh(
        core_axis_name="c", subcore_axis_name="s", num_cores=NC, num_subcores=NS
    )


def _fill_rows(rows_ref, nrows, width, value):
    vec = jnp.full((16,), value, jnp.float32)

    @pl.loop(0, nrows)
    def _(r):
        @pl.loop(0, width // 16)
        def _(l):
            rows_ref.at[r][pl.ds(l * 16, 16)] = vec


def _zero_acc_slice(rows, acc, sid):
    row0 = sid * ROWS_A
    off = 0
    for ln in Z_LENS:
        pltpu.sync_copy(rows.at[pl.ds(0, ln)], acc.at[pl.ds(row0 + off, ln)])
        off += ln

    @pl.when(sid == 0)
    def _():
        pltpu.sync_copy(rows.at[pl.ds(0, ZTAIL)], acc.at[pl.ds(TAIL0, ZTAIL)])


def _write_back(acc, out_hbm, cid, sid):
    row0 = sid * ROWS_A
    pltpu.sync_copy(acc.at[pl.ds(row0, ROWS_A)],
                    out_hbm.at[cid, pl.ds(row0, ROWS_A)])

    @pl.when(sid == 0)
    def _():
        pltpu.sync_copy(acc.at[pl.ds(TAIL0, WTAIL)],
                        out_hbm.at[cid, pl.ds(TAIL0, WTAIL)])


@functools.cache
def _make_deg_kernel():
    @functools.partial(
        pl.kernel,
        out_type=jax.ShapeDtypeStruct((NC, N, DEG_W), jnp.float32),
        mesh=_sc_mesh(),
        scratch_types=(
            [pltpu.VMEM((CHUNK,), jnp.int32) for _ in range(NIDX)]
            + [pltpu.VMEM((CHUNK, DEG_W), jnp.float32),
               pltpu.VMEM_SHARED((NROWS_ACC, DEG_W), jnp.float32)]
            + [pltpu.SemaphoreType.DMA] * (NIDX + NBUF)
        ),
    )
    def deg_kernel(dst_hbm, out_hbm, *sc):
        idx_d = sc[:NIDX]
        ones = sc[NIDX]
        acc = sc[NIDX + 1]
        sem_i = sc[NIDX + 2:NIDX + 2 + NIDX]
        sem_s = sc[NIDX + 2 + NIDX:]
        cid = lax.axis_index("c")
        sid = lax.axis_index("s")
        tid = cid * NS + sid
        e0 = tid * EPT_PAD

        _fill_rows(ones, CHUNK, DEG_W, 0.0)
        _zero_acc_slice(ones, acc, sid)
        plsc.subcore_barrier()
        _fill_rows(ones, CHUNK, DEG_W, 1.0)

        def idx_start(k, s8):
            pltpu.async_copy(dst_hbm.at[pl.ds(e0 + k * CHUNK, CHUNK)],
                             idx_d[s8], sem_i[s8])

        def idx_wait(s8):
            pltpu.make_async_copy(dst_hbm.at[pl.ds(0, CHUNK)], idx_d[s8],
                                  sem_i[s8]).wait()

        def sc_start(s4, s8):
            pltpu.async_copy(ones, acc.at[idx_d[s8]], sem_s[s4], add=True)

        def sc_wait(s4, s8):
            pltpu.make_async_copy(ones, acc.at[idx_d[s8]], sem_s[s4]).wait()

        def step(k, s4, s8, ph1=True, ph4=True):
            if ph1:
                sc_wait(s4, (s8 + 4) % NIDX)
            idx_wait(s8)
            sc_start(s4, s8)
            if ph4:
                idx_start(k + 4, (s8 + 4) % NIDX)

        for j in range(4):
            idx_start(j, j)
        for k in range(4):
            step(k, k % NBUF, k % NIDX, ph1=False)

        @pl.loop(0, (STEADY1 - STEADY0) // NIDX)
        def _(m):
            base = STEADY0 + m * NIDX
            for u in range(NIDX):
                step(base + u, (STEADY0 + u) % NBUF, (STEADY0 + u) % NIDX)

        for k in range(STEADY1, CPT):
            step(k, k % NBUF, k % NIDX, ph4=False)
        for k in range(CPT - 4, CPT):
            sc_wait(k % NBUF, k % NIDX)

        plsc.subcore_barrier()
        _write_back(acc, out_hbm, cid, sid)

    return deg_kernel


@functools.cache
def _make_agg_kernel():
    @functools.partial(
        pl.kernel,
        out_type=jax.ShapeDtypeStruct((NC, N, D), jnp.float32),
        mesh=_sc_mesh(),
        scratch_types=(
            [pltpu.VMEM((CHUNK,), jnp.int32) for _ in range(2 * A_NIDX)]
            + [pltpu.VMEM((CHUNK, D), jnp.float32) for _ in range(A_NBUF)]
            + [pltpu.VMEM_SHARED((NROWS_ACC, D), jnp.float32)]
            + [pltpu.SemaphoreType.DMA] * (A_NIDX + 2 * A_NBUF)
        ),
    )
    def agg_kernel(g_hbm, src_hbm, dst_hbm, out_hbm, *sc):
        idx_s = sc[:A_NIDX]
        idx_d = sc[A_NIDX:2 * A_NIDX]
        rows = sc[2 * A_NIDX:2 * A_NIDX + A_NBUF]
        acc = sc[2 * A_NIDX + A_NBUF]
        sems = sc[2 * A_NIDX + A_NBUF + 1:]
        sem_i = sems[:A_NIDX]
        sem_g = sems[A_NIDX:A_NIDX + A_NBUF]
        sem_s = sems[A_NIDX + A_NBUF:]
        cid = lax.axis_index("c")
        sid = lax.axis_index("s")
        tid = cid * NS + sid
        e0 = tid * EPT_PAD

        _fill_rows(rows[0], CHUNK, D, 0.0)
        _zero_acc_slice(rows[0], acc, sid)
        plsc.subcore_barrier()

        def idx_start(k, s6):
            b = e0 + k * CHUNK
            pltpu.async_copy(src_hbm.at[pl.ds(b, CHUNK)], idx_s[s6], sem_i[s6])
            pltpu.async_copy(dst_hbm.at[pl.ds(b, CHUNK)], idx_d[s6], sem_i[s6])

        def idx_wait(s6):
            pltpu.make_async_copy(src_hbm.at[pl.ds(0, CHUNK)], idx_s[s6],
                                  sem_i[s6]).wait()
            pltpu.make_async_copy(dst_hbm.at[pl.ds(0, CHUNK)], idx_d[s6],
                                  sem_i[s6]).wait()

        def g_start(s3, s6):
            pltpu.async_copy(g_hbm.at[idx_s[s6]], rows[s3], sem_g[s3])

        def g_wait(s3, s6):
            pltpu.make_async_copy(g_hbm.at[idx_s[s6]], rows[s3],
                                  sem_g[s3]).wait()

        def scat(s3, s6):
            pltpu.sync_copy(rows[s3], acc.at[idx_d[s6]], add=True)

        def step(k, s3, s6, ph1, ph2, ph3, ph4):
            del ph1
            if ph2:
                idx_wait(s6)
                g_start(s3, s6)
            if ph3:
                g_wait((s3 + 1) % A_NBUF, (s6 + 4) % A_NIDX)
                scat((s3 + 1) % A_NBUF, (s6 + 4) % A_NIDX)
            if ph4:
                idx_start(k + 3, (s6 + 3) % A_NIDX)

        def py_step(k):
            step(k, k % A_NBUF, k % A_NIDX,
                 ph1=k >= A_STEADY0,
                 ph2=k < CPT,
                 ph3=2 <= k < CPT + 2,
                 ph4=k + 3 < CPT)

        for j in range(A_STEADY0):
            idx_start(j, j)
        for k in range(A_STEADY0):
            py_step(k)

        n_steady = (CPT - A_NBUF - A_STEADY0) // A_NIDX

        @pl.loop(0, n_steady)
        def _(m):
            base = A_STEADY0 + m * A_NIDX
            for u in range(A_NIDX):
                step(base + u, (A_STEADY0 + u) % A_NBUF,
                     (A_STEADY0 + u) % A_NIDX, True, True, True, True)

        for k in range(A_STEADY0 + n_steady * A_NIDX, CPT + 2):
            py_step(k)

        plsc.subcore_barrier()
        _write_back(acc, out_hbm, cid, sid)

    return agg_kernel


def _m1_body(x_ref, w_ref, degp_ref, g1_ref, dinv_ref):
    deg = degp_ref[0, :, 0:1] + degp_ref[1, :, 0:1] + 1.0
    dinv = lax.rsqrt(deg)
    dinv_ref[...] = dinv
    h = jnp.dot(x_ref[...], w_ref[...], preferred_element_type=jnp.float32)
    g1_ref[...] = h * dinv


def _m2_body(p_ref, g1_ref, dinv_ref, b1_ref, u_ref):
    s = p_ref[0] + p_ref[1] + g1_ref[...]
    dinv = dinv_ref[...]
    h1 = jnp.maximum(s * dinv + b1_ref[...], 0.0)
    u_ref[...] = h1 * dinv


def _e3_body(p_ref, u_ref, dinv_ref, b2_ref, w2_ref, out_ref):
    s = p_ref[0] + p_ref[1] + u_ref[...]
    h2 = jnp.dot(s, w2_ref[...], preferred_element_type=jnp.float32)
    out_ref[...] = h2 * dinv_ref[...] + b2_ref[...]


_m1 = pl.pallas_call(
    _m1_body,
    out_shape=(jax.ShapeDtypeStruct((N, D), jnp.float32),
               jax.ShapeDtypeStruct((N, 1), jnp.float32)),
)
_m2 = pl.pallas_call(
    _m2_body,
    out_shape=jax.ShapeDtypeStruct((N, D), jnp.float32),
)
_e3 = pl.pallas_call(
    _e3_body,
    out_shape=jax.ShapeDtypeStruct((N, D_OUT_PAD), jnp.float32),
)


@jax.jit
def _run(x, edge_index, W1, b1, W2, b2):
    ei = edge_index.astype(jnp.int32)
    src = jnp.pad(ei[0].reshape(NT, EPT), ((0, 0), (0, EPT_PAD - EPT)),
                  constant_values=0).reshape(-1)
    dst = jnp.pad(ei[1].reshape(NT, EPT), ((0, 0), (0, EPT_PAD - EPT)),
                  constant_values=N).reshape(-1)
    w2p = jnp.pad(W2, ((0, 0), (0, D_OUT_PAD - D_OUT)))
    b1r = b1.reshape(1, D)
    b2r = jnp.pad(b2, (0, D_OUT_PAD - D_OUT)).reshape(1, D_OUT_PAD)

    degp = _make_deg_kernel()(dst)
    g1, dinv = _m1(x, W1, degp)
    p1 = _make_agg_kernel()(g1, src, dst)
    u = _m2(p1, g1, dinv, b1r)
    p2 = _make_agg_kernel()(u, src, dst)
    out = _e3(p2, u, dinv, b2r, w2p)
    return out[:, :D_OUT]


def kernel(x, edge_index, W1, b1, W2, b2):
    return _run(x, edge_index, W1, b1, W2, b2)


# speedup vs baseline: 1.0039x; 1.0039x over previous
import functools

import jax
import jax.numpy as jnp
from jax import lax
from jax.experimental import pallas as pl
from jax.experimental.pallas import tpu as pltpu
from jax.experimental.pallas import tpu_sc as plsc

N = 10000
E = 320000
D = 128
DEG_W = 16
D_OUT = 40
D_OUT_PAD = 48

NC = 2
NS = 16
NT = NC * NS
CHUNK = 128
CPT = 80
EPT = E // NT
EPT_PAD = CPT * CHUNK

NROWS_ACC = N + 16
ROWS_A = 624
TAIL0 = NS * ROWS_A
WTAIL = N - TAIL0
ZTAIL = NROWS_ACC - TAIL0
Z_LENS = (128, 128, 128, 128, 112)

NBUF = 4
NIDX = 8
STEADY0 = 4
STEADY1 = CPT - 4

A_NBUF = 3
A_NIDX = 6
A_STEADY0 = 3


def _sc_mesh():
    return plsc.VectorSubcoreMesh(
        core_axis_name="c", subcore_axis_name="s", num_cores=NC, num_subcores=NS
    )


def _fill_rows(rows_ref, nrows, width, value):
    vec = jnp.full((16,), value, jnp.float32)

    @pl.loop(0, nrows)
    def _(r):
        @pl.loop(0, width // 16)
        def _(l):
            rows_ref.at[r][pl.ds(l * 16, 16)] = vec


def _zero_acc_slice(rows, acc, sid):
    row0 = sid * ROWS_A
    off = 0
    for ln in Z_LENS:
        pltpu.sync_copy(rows.at[pl.ds(0, ln)], acc.at[pl.ds(row0 + off, ln)])
        off += ln

    @pl.when(sid == 0)
    def _():
        pltpu.sync_copy(rows.at[pl.ds(0, ZTAIL)], acc.at[pl.ds(TAIL0, ZTAIL)])


def _write_back(acc, out_hbm, cid, sid):
    row0 = sid * ROWS_A
    pltpu.sync_copy(acc.at[pl.ds(row0, ROWS_A)],
                    out_hbm.at[cid, pl.ds(row0, ROWS_A)])

    @pl.when(sid == 0)
    def _():
        pltpu.sync_copy(acc.at[pl.ds(TAIL0, WTAIL)],
                        out_hbm.at[cid, pl.ds(TAIL0, WTAIL)])


@functools.cache
def _make_deg_kernel():
    @functools.partial(
        pl.kernel,
        out_type=jax.ShapeDtypeStruct((NC, N, DEG_W), jnp.float32),
        mesh=_sc_mesh(),
        scratch_types=(
            [pltpu.VMEM((CHUNK,), jnp.int32) for _ in range(NIDX)]
            + [pltpu.VMEM((CHUNK, DEG_W), jnp.float32),
               pltpu.VMEM_SHARED((NROWS_ACC, DEG_W), jnp.float32)]
            + [pltpu.SemaphoreType.DMA] * (NIDX + NBUF)
        ),
    )
    def deg_kernel(dst_hbm, out_hbm, *sc):
        idx_d = sc[:NIDX]
        ones = sc[NIDX]
        acc = sc[NIDX + 1]
        sem_i = sc[NIDX + 2:NIDX + 2 + NIDX]
        sem_s = sc[NIDX + 2 + NIDX:]
        cid = lax.axis_index("c")
        sid = lax.axis_index("s")
        tid = cid * NS + sid
        e0 = tid * EPT_PAD

        _fill_rows(ones, CHUNK, DEG_W, 0.0)
        _zero_acc_slice(ones, acc, sid)
        plsc.subcore_barrier()
        _fill_rows(ones, CHUNK, DEG_W, 1.0)

        def idx_start(k, s8):
            pltpu.async_copy(dst_hbm.at[pl.ds(e0 + k * CHUNK, CHUNK)],
                             idx_d[s8], sem_i[s8])

        def idx_wait(s8):
            pltpu.make_async_copy(dst_hbm.at[pl.ds(0, CHUNK)], idx_d[s8],
                                  sem_i[s8]).wait()

        def sc_start(s4, s8):
            pltpu.async_copy(ones, acc.at[idx_d[s8]], sem_s[s4], add=True)

        def sc_wait(s4, s8):
            pltpu.make_async_copy(ones, acc.at[idx_d[s8]], sem_s[s4]).wait()

        def step(k, s4, s8, ph1=True, ph4=True):
            if ph1:
                sc_wait(s4, (s8 + 4) % NIDX)
            idx_wait(s8)
            sc_start(s4, s8)
            if ph4:
                idx_start(k + 4, (s8 + 4) % NIDX)

        for j in range(4):
            idx_start(j, j)
        for k in range(4):
            step(k, k % NBUF, k % NIDX, ph1=False)

        @pl.loop(0, (STEADY1 - STEADY0) // NIDX)
        def _(m):
            base = STEADY0 + m * NIDX
            for u in range(NIDX):
                step(base + u, (STEADY0 + u) % NBUF, (STEADY0 + u) % NIDX)

        for k in range(STEADY1, CPT):
            step(k, k % NBUF, k % NIDX, ph4=False)
        for k in range(CPT - 4, CPT):
            sc_wait(k % NBUF, k % NIDX)

        plsc.subcore_barrier()
        _write_back(acc, out_hbm, cid, sid)

    return deg_kernel


@functools.cache
def _make_agg_kernel():
    @functools.partial(
        pl.kernel,
        out_type=jax.ShapeDtypeStruct((NC, N, D), jnp.float32),
        mesh=_sc_mesh(),
        scratch_types=(
            [pltpu.VMEM((CHUNK,), jnp.int32) for _ in range(2 * A_NIDX)]
            + [pltpu.VMEM((CHUNK, D), jnp.float32) for _ in range(A_NBUF)]
            + [pltpu.VMEM_SHARED((NROWS_ACC, D), jnp.float32)]
            + [pltpu.SemaphoreType.DMA] * (A_NIDX + 2 * A_NBUF)
        ),
    )
    def agg_kernel(g_hbm, src_hbm, dst_hbm, out_hbm, *sc):
        idx_s = sc[:A_NIDX]
        idx_d = sc[A_NIDX:2 * A_NIDX]
        rows = sc[2 * A_NIDX:2 * A_NIDX + A_NBUF]
        acc = sc[2 * A_NIDX + A_NBUF]
        sems = sc[2 * A_NIDX + A_NBUF + 1:]
        sem_i = sems[:A_NIDX]
        sem_g = sems[A_NIDX:A_NIDX + A_NBUF]
        sem_s = sems[A_NIDX + A_NBUF:]
        cid = lax.axis_index("c")
        sid = lax.axis_index("s")
        tid = cid * NS + sid
        e0 = tid * EPT_PAD

        _fill_rows(rows[0], CHUNK, D, 0.0)
        _zero_acc_slice(rows[0], acc, sid)
        plsc.subcore_barrier()

        def idx_start(k, s6):
            b = e0 + k * CHUNK
            pltpu.async_copy(src_hbm.at[pl.ds(b, CHUNK)], idx_s[s6], sem_i[s6])
            pltpu.async_copy(dst_hbm.at[pl.ds(b, CHUNK)], idx_d[s6], sem_i[s6])

        def idx_wait(s6):
            pltpu.make_async_copy(src_hbm.at[pl.ds(0, CHUNK)], idx_s[s6],
                                  sem_i[s6]).wait()
            pltpu.make_async_copy(dst_hbm.at[pl.ds(0, CHUNK)], idx_d[s6],
                                  sem_i[s6]).wait()

        def g_start(s3, s6):
            pltpu.async_copy(g_hbm.at[idx_s[s6]], rows[s3], sem_g[s3])

        def g_wait(s3, s6):
            pltpu.make_async_copy(g_hbm.at[idx_s[s6]], rows[s3],
                                  sem_g[s3]).wait()

        def scat(s3, s6):
            pltpu.sync_copy(rows[s3], acc.at[idx_d[s6]], add=True)

        def step(k, s3, s6, ph1, ph2, ph3, ph4):
            del ph1
            if ph2:
                idx_wait(s6)
                g_start(s3, s6)
            if ph3:
                g_wait((s3 + 1) % A_NBUF, (s6 + 4) % A_NIDX)
                scat((s3 + 1) % A_NBUF, (s6 + 4) % A_NIDX)
            if ph4:
                idx_start(k + 3, (s6 + 3) % A_NIDX)

        def py_step(k):
            step(k, k % A_NBUF, k % A_NIDX,
                 ph1=k >= A_STEADY0,
                 ph2=k < CPT,
                 ph3=2 <= k < CPT + 2,
                 ph4=k + 3 < CPT)

        for j in range(A_STEADY0):
            idx_start(j, j)
        for k in range(A_STEADY0):
            py_step(k)

        n_steady = (CPT - A_NBUF - A_STEADY0) // A_NIDX

        @pl.loop(0, n_steady)
        def _(m):
            base = A_STEADY0 + m * A_NIDX
            for u in range(A_NIDX):
                step(base + u, (A_STEADY0 + u) % A_NBUF,
                     (A_STEADY0 + u) % A_NIDX, True, True, True, True)

        for k in range(A_STEADY0 + n_steady * A_NIDX, CPT + 2):
            py_step(k)

        plsc.subcore_barrier()
        _write_back(acc, out_hbm, cid, sid)

    return agg_kernel


def _m1_body(x_ref, w_ref, degp_ref, g1_ref, dinv_ref):
    deg = degp_ref[0, :, 0:1] + degp_ref[1, :, 0:1] + 1.0
    dinv = lax.rsqrt(deg)
    dinv_ref[...] = dinv
    h = jnp.dot(x_ref[...], w_ref[...], preferred_element_type=jnp.float32)
    g1_ref[...] = h * dinv


def _m2_body(p_ref, g1_ref, dinv_ref, b1_ref, u_ref):
    s = p_ref[0] + p_ref[1] + g1_ref[...]
    dinv = dinv_ref[...]
    h1 = jnp.maximum(s * dinv + b1_ref[...], 0.0)
    u_ref[...] = h1 * dinv


def _e3_body(p_ref, u_ref, dinv_ref, b2_ref, w2_ref, out_ref):
    s = p_ref[0] + p_ref[1] + u_ref[...]
    h2 = jnp.dot(s, w2_ref[...], preferred_element_type=jnp.float32)
    out_ref[...] = h2 * dinv_ref[...] + b2_ref[...]


_m1 = pl.pallas_call(
    _m1_body,
    out_shape=(jax.ShapeDtypeStruct((N, D), jnp.float32),
               jax.ShapeDtypeStruct((N, 1), jnp.float32)),
)
_m2 = pl.pallas_call(
    _m2_body,
    out_shape=jax.ShapeDtypeStruct((N, D), jnp.float32),
)
_e3 = pl.pallas_call(
    _e3_body,
    out_shape=jax.ShapeDtypeStruct((N, D_OUT_PAD), jnp.float32),
)


@jax.jit
def _run(x, edge_index, W1, b1, W2, b2):
    ei = edge_index.astype(jnp.int32)
    src = jnp.pad(ei[0].reshape(NT, EPT), ((0, 0), (0, EPT_PAD - EPT)),
                  constant_values=0).reshape(-1)
    trash = (N + (jnp.arange(NT, dtype=jnp.int32) % NS))[:, None]
    dst = jnp.concatenate(
        [ei[1].reshape(NT, EPT),
         jnp.broadcast_to(trash, (NT, EPT_PAD - EPT))], axis=1).reshape(-1)
    w2p = jnp.pad(W2, ((0, 0), (0, D_OUT_PAD - D_OUT)))
    b1r = b1.reshape(1, D)
    b2r = jnp.pad(b2, (0, D_OUT_PAD - D_OUT)).reshape(1, D_OUT_PAD)

    degp = _make_deg_kernel()(dst)
    g1, dinv = _m1(x, W1, degp)
    p1 = _make_agg_kernel()(g1, src, dst)
    u = _m2(p1, g1, dinv, b1r)
    p2 = _make_agg_kernel()(u, src, dst)
    out = _e3(p2, u, dinv, b2r, w2p)
    return out[:, :D_OUT]


def kernel(x, edge_index, W1, b1, W2, b2):
    return _run(x, edge_index, W1, b1, W2, b2)
